# baseline (speedup 1.0000x reference)
import functools
import math

import jax
import jax.numpy as jnp
from jax import lax
from jax.experimental import pallas as pl
from jax.experimental.pallas import tpu as pltpu
from jax.experimental.pallas import tpu_sc as plsc

N = 10000
NPAD = 10016
ROWS_PER_TILE = NPAD // 16
D = 32
CHUNK = 128
NW = 32
TILE_E = 512


def _mesh():
    return plsc.VectorSubcoreMesh(core_axis_name="c", subcore_axis_name="s")


_SC_PARAMS = pltpu.CompilerParams(use_tc_tiling_on_sc=False)


def _gather_call(table, idx, ep):
    n_chunks = ep // NW // CHUNK

    @functools.partial(
        pl.kernel,
        out_type=jax.ShapeDtypeStruct((ep, D), jnp.float32),
        mesh=_mesh(),
        compiler_params=_SC_PARAMS,
        scratch_types=[
            pltpu.VMEM((CHUNK,), jnp.int32),
            pltpu.VMEM((CHUNK, D), jnp.float32),
            pltpu.SemaphoreType.DMA,
        ],
    )
    def k(table_hbm, idx_hbm, out_hbm, idx_v, rows_v, sem):
        c = lax.axis_index("c")
        s = lax.axis_index("s")
        wid = s * 2 + c
        base = wid * (ep // NW)

        def body(j, _):
            off = base + j * CHUNK
            pltpu.sync_copy(idx_hbm.at[pl.ds(off, CHUNK)], idx_v)
            pltpu.async_copy(table_hbm.at[idx_v], rows_v, sem).wait()
            pltpu.sync_copy(rows_v, out_hbm.at[pl.ds(off, CHUNK)])
            return 0

        lax.fori_loop(0, n_chunks, body, 0)

    return k(table, idx)


def _scatter_call(msg, dst, ep):
    n_chunks = ep // NW // CHUNK

    @functools.partial(
        pl.kernel,
        out_type=jax.ShapeDtypeStruct((2, NPAD, D), jnp.float32),
        mesh=_mesh(),
        compiler_params=_SC_PARAMS,
        scratch_types=[
            pltpu.VMEM((CHUNK,), jnp.int32),
            pltpu.VMEM((CHUNK, D), jnp.float32),
            pltpu.VMEM((ROWS_PER_TILE, D), jnp.float32),
            pltpu.VMEM_SHARED((NPAD, D), jnp.float32),
            pltpu.SemaphoreType.DMA,
        ],
    )
    def k(msg_hbm, dst_hbm, out_hbm, idx_v, msg_v, zero_v, acc_sh, sem):
        c = lax.axis_index("c")
        s = lax.axis_index("s")
        wid = s * 2 + c
        base = wid * (ep // NW)

        z16 = jnp.zeros((16,), jnp.float32)

        def zrow(i, _):
            zero_v[i, 0:16] = z16
            zero_v[i, 16:32] = z16
            return 0

        lax.fori_loop(0, ROWS_PER_TILE, zrow, 0)
        pltpu.sync_copy(zero_v, acc_sh.at[pl.ds(s * ROWS_PER_TILE, ROWS_PER_TILE)])
        plsc.subcore_barrier()

        def body(j, _):
            off = base + j * CHUNK
            pltpu.sync_copy(dst_hbm.at[pl.ds(off, CHUNK)], idx_v)
            pltpu.sync_copy(msg_hbm.at[pl.ds(off, CHUNK)], msg_v)
            pltpu.sync_copy(msg_v, acc_sh.at[idx_v], add=True)
            return 0

        lax.fori_loop(0, n_chunks, body, 0)
        plsc.subcore_barrier()
        pltpu.sync_copy(
            acc_sh.at[pl.ds(s * ROWS_PER_TILE, ROWS_PER_TILE)],
            out_hbm.at[c, pl.ds(s * ROWS_PER_TILE, ROWS_PER_TILE)],
        )

    return k(msg, dst)


def _count_call(dst, ep):
    n_chunks = ep // NW // CHUNK

    @functools.partial(
        pl.kernel,
        out_type=jax.ShapeDtypeStruct((2, NPAD, 16), jnp.float32),
        mesh=_mesh(),
        compiler_params=_SC_PARAMS,
        scratch_types=[
            pltpu.VMEM((CHUNK,), jnp.int32),
            pltpu.VMEM((CHUNK, 16), jnp.float32),
            pltpu.VMEM((ROWS_PER_TILE, 16), jnp.float32),
            pltpu.VMEM_SHARED((NPAD, 16), jnp.float32),
        ],
    )
    def k(dst_hbm, out_hbm, idx_v, ones_v, zero_v, acc_sh):
        c = lax.axis_index("c")
        s = lax.axis_index("s")
        wid = s * 2 + c
        base = wid * (ep // NW)

        z16 = jnp.zeros((16,), jnp.float32)
        o16 = jnp.ones((16,), jnp.float32)

        def fill(i, _):
            zero_v[i, 0:16] = z16
            return 0

        def fill1(i, _):
            ones_v[i, 0:16] = o16
            return 0

        lax.fori_loop(0, ROWS_PER_TILE, fill, 0)
        lax.fori_loop(0, CHUNK, fill1, 0)
        pltpu.sync_copy(zero_v, acc_sh.at[pl.ds(s * ROWS_PER_TILE, ROWS_PER_TILE)])
        plsc.subcore_barrier()

        def body(j, _):
            off = base + j * CHUNK
            pltpu.sync_copy(dst_hbm.at[pl.ds(off, CHUNK)], idx_v)
            pltpu.sync_copy(ones_v, acc_sh.at[idx_v], add=True)
            return 0

        lax.fori_loop(0, n_chunks, body, 0)
        plsc.subcore_barrier()
        pltpu.sync_copy(
            acc_sh.at[pl.ds(s * ROWS_PER_TILE, ROWS_PER_TILE)],
            out_hbm.at[c, pl.ds(s * ROWS_PER_TILE, ROWS_PER_TILE)],
        )

    return k(dst)


def _edge_body(ea_ref, xs_ref, w1_ref, b1_ref, w2r_ref, bm_ref, out_ref):
    g = jax.nn.gelu(
        jnp.dot(ea_ref[...], w1_ref[...], preferred_element_type=jnp.float32)
        + b1_ref[...]
    )
    xs = xs_ref[...]
    z = jnp.concatenate([g[:, h : h + 1] * xs for h in range(D)], axis=1)
    out_ref[...] = (
        jnp.dot(z, w2r_ref[...], preferred_element_type=jnp.float32)
        + jnp.dot(xs, bm_ref[...], preferred_element_type=jnp.float32)
    )


def _edge_call(ea, xs, w1p, b1, w2r, bm, ep):
    grid = ep // TILE_E
    return pl.pallas_call(
        _edge_body,
        grid=(grid,),
        in_specs=[
            pl.BlockSpec((TILE_E, 8), lambda i: (i, 0)),
            pl.BlockSpec((TILE_E, D), lambda i: (i, 0)),
            pl.BlockSpec((8, D), lambda i: (0, 0)),
            pl.BlockSpec((1, D), lambda i: (0, 0)),
            pl.BlockSpec((D * D, D), lambda i: (0, 0)),
            pl.BlockSpec((D, D), lambda i: (0, 0)),
        ],
        out_specs=pl.BlockSpec((TILE_E, D), lambda i: (i, 0)),
        out_shape=jax.ShapeDtypeStruct((ep, D), jnp.float32),
    )(ea, xs, w1p, b1, w2r, bm)


def _node_body(x_ref, parts_ref, cnt_ref, rw_ref, rb_ref, extra_ref, out_ref):
    agg = parts_ref[0] + parts_ref[1]
    cnt = cnt_ref[0, :, 0:1] + cnt_ref[1, :, 0:1]
    mean = agg / jnp.maximum(cnt, 1.0)
    h = (
        jnp.dot(x_ref[...], rw_ref[...], preferred_element_type=jnp.float32)
        + rb_ref[...]
        + mean
    )
    out_ref[...] = jax.nn.gelu(h) + extra_ref[...]


def _node_call(x, parts, cnt, rw, rb, extra):
    return pl.pallas_call(
        _node_body,
        out_shape=jax.ShapeDtypeStruct((NPAD, D), jnp.float32),
    )(x, parts, cnt, rw, rb, extra)


def _proj_body(xin_ref, w1_ref, b1_ref, w2_ref, b2_ref, out_ref):
    h = jax.nn.gelu(
        jnp.dot(xin_ref[...], w1_ref[...], preferred_element_type=jnp.float32)
        + b1_ref[...]
    )
    out_ref[...] = jax.nn.gelu(
        jnp.dot(h, w2_ref[...], preferred_element_type=jnp.float32) + b2_ref[...]
    )


def _dec_body(a_ref, b_ref, w1_ref, b1_ref, w2_ref, b2_ref, out_ref):
    h = jax.nn.gelu(
        jnp.dot(a_ref[...] + b_ref[...], w1_ref[...], preferred_element_type=jnp.float32)
        + b1_ref[...]
    )
    out_ref[...] = (
        jnp.dot(h, w2_ref[...], preferred_element_type=jnp.float32) + b2_ref[...]
    )


def _prep_edges(edge_index, edge_attr):
    e = edge_attr.shape[0]
    ep = math.ceil(e / (NW * CHUNK)) * (NW * CHUNK)
    src = jnp.concatenate(
        [edge_index[0], jnp.zeros((ep - e,), edge_index.dtype)]
    ).astype(jnp.int32)
    dst = jnp.concatenate(
        [edge_index[1], jnp.full((ep - e,), N, edge_index.dtype)]
    ).astype(jnp.int32)
    ea = jnp.pad(edge_attr, ((0, ep - e), (0, 8 - edge_attr.shape[1])))
    return src, dst, ea, ep


def _prep_gno(p):
    kp = p["kernel"]
    return dict(
        w1p=jnp.pad(kp["w1"], ((0, 8 - kp["w1"].shape[0]), (0, 0))),
        b1=kp["b1"][None, :],
        w2r=kp["w2"].reshape(D * D, D),
        bm=kp["b2"].reshape(D, D),
        rw=p["root_w"],
        rb=p["root_b"][None, :],
    )


def _gno_layer(x, src, dst, ea, ep, cnt, w, extra):
    xs = _gather_call(x, src, ep)
    msg = _edge_call(ea, xs, w["w1p"], w["b1"], w["w2r"], w["bm"], ep)
    parts = _scatter_call(msg, dst, ep)
    return _node_call(x, parts, cnt, w["rw"], w["rb"], extra)


def kernel(nodes, grid, edge_index_1, edge_index_2, edge_index_3,
           edge_attr_1, edge_attr_2, edge_attr_3, batch_size, image_size, params):
    src1, dst1, ea1, ep1 = _prep_edges(edge_index_1, edge_attr_1)
    src2, dst2, ea2, ep2 = _prep_edges(edge_index_2, edge_attr_2)
    src3, dst3, ea3, ep3 = _prep_edges(edge_index_3, edge_attr_3)

    w11 = _prep_gno(params["k11"][0])
    w12 = _prep_gno(params["k12"][0])
    w22 = _prep_gno(params["k22"][0])
    w23 = _prep_gno(params["k23"][0])
    w33 = _prep_gno(params["k33"][0])
    w32 = _prep_gno(params["k32"][0])
    w21 = _prep_gno(params["k21"][0])

    xin = jnp.pad(jnp.concatenate([nodes, grid], axis=1), ((0, NPAD - N), (0, 4)))
    pp = params["projector"]
    x0 = pl.pallas_call(
        _proj_body,
        out_shape=jax.ShapeDtypeStruct((NPAD, D), jnp.float32),
    )(
        xin,
        jnp.pad(pp["w1"], ((0, 4), (0, 0))),
        pp["b1"][None, :],
        pp["w2"],
        pp["b2"][None, :],
    )

    cnt1 = _count_call(dst1, ep1)
    cnt2 = _count_call(dst2, ep2)
    cnt3 = _count_call(dst3, ep3)
    zero_extra = jnp.zeros((NPAD, D), jnp.float32)

    xs1 = _gather_call(x0, src1, ep1)
    msg11 = _edge_call(ea1, xs1, w11["w1p"], w11["b1"], w11["w2r"], w11["bm"], ep1)
    msg12 = _edge_call(ea1, xs1, w12["w1p"], w12["b1"], w12["w2r"], w12["bm"], ep1)
    n11 = _node_call(x0, _scatter_call(msg11, dst1, ep1), cnt1,
                     w11["rw"], w11["rb"], zero_extra)
    n12 = _node_call(x0, _scatter_call(msg12, dst1, ep1), cnt1,
                     w12["rw"], w12["rb"], zero_extra)

    xs2 = _gather_call(n12, src2, ep2)
    msg22 = _edge_call(ea2, xs2, w22["w1p"], w22["b1"], w22["w2r"], w22["bm"], ep2)
    msg23 = _edge_call(ea2, xs2, w23["w1p"], w23["b1"], w23["w2r"], w23["bm"], ep2)
    n22 = _node_call(n12, _scatter_call(msg22, dst2, ep2), cnt2,
                     w22["rw"], w22["rb"], zero_extra)
    n23 = _node_call(n12, _scatter_call(msg23, dst2, ep2), cnt2,
                     w23["rw"], w23["rb"], zero_extra)

    n33 = _gno_layer(n23, src3, dst3, ea3, ep3, cnt3, w33, zero_extra)
    s32 = _gno_layer(n33, src2, dst2, ea2, ep2, cnt2, w32, n22)
    n21 = _gno_layer(s32, src1, dst1, ea1, ep1, cnt1, w21, zero_extra)

    dp = params["decoder"]
    out = pl.pallas_call(
        _dec_body,
        out_shape=jax.ShapeDtypeStruct((NPAD, 1), jnp.float32),
    )(n21, n11, dp["w1"], dp["b1"][None, :], dp["w2"], dp["b2"][None, :])
    return out[:N]


# speedup vs baseline: 2.1902x; 2.1902x over previous
import functools
import math

import jax
import jax.numpy as jnp
from jax import lax
from jax.experimental import pallas as pl
from jax.experimental.pallas import tpu as pltpu
from jax.experimental.pallas import tpu_sc as plsc

N = 10000
NPAD = 10016
ROWS_PER_TILE = NPAD // 16
D = 32
CHUNK = 128
NW = 32
TILE_E = 512


def _mesh():
    return plsc.VectorSubcoreMesh(core_axis_name="c", subcore_axis_name="s")


_SC_PARAMS = pltpu.CompilerParams(use_tc_tiling_on_sc=False)


def _gather_call(table, idx, ep):
    n_chunks = ep // NW // CHUNK

    @functools.partial(
        pl.kernel,
        out_type=jax.ShapeDtypeStruct((ep, D), jnp.float32),
        mesh=_mesh(),
        compiler_params=_SC_PARAMS,
        scratch_types=[
            pltpu.VMEM((CHUNK,), jnp.int32),
            pltpu.VMEM((CHUNK, D), jnp.float32),
            pltpu.SemaphoreType.DMA,
        ],
    )
    def k(table_hbm, idx_hbm, out_hbm, idx_v, rows_v, sem):
        c = lax.axis_index("c")
        s = lax.axis_index("s")
        wid = s * 2 + c
        base = wid * (ep // NW)

        def body(j, _):
            off = base + j * CHUNK
            pltpu.sync_copy(idx_hbm.at[pl.ds(off, CHUNK)], idx_v)
            pltpu.async_copy(table_hbm.at[idx_v], rows_v, sem).wait()
            pltpu.sync_copy(rows_v, out_hbm.at[pl.ds(off, CHUNK)])
            return 0

        lax.fori_loop(0, n_chunks, body, 0)

    return k(table, idx)


def _scatter_call(msg, dst, ep):
    n_chunks = ep // NW // CHUNK

    @functools.partial(
        pl.kernel,
        out_type=jax.ShapeDtypeStruct((2, NPAD, D), jnp.float32),
        mesh=_mesh(),
        compiler_params=_SC_PARAMS,
        scratch_types=[
            pltpu.VMEM((CHUNK,), jnp.int32),
            pltpu.VMEM((CHUNK, D), jnp.float32),
            pltpu.VMEM((ROWS_PER_TILE, D), jnp.float32),
            pltpu.VMEM_SHARED((NPAD, D), jnp.float32),
            pltpu.SemaphoreType.DMA,
        ],
    )
    def k(msg_hbm, dst_hbm, out_hbm, idx_v, msg_v, zero_v, acc_sh, sem):
        c = lax.axis_index("c")
        s = lax.axis_index("s")
        wid = s * 2 + c
        base = wid * (ep // NW)

        z16 = jnp.zeros((16,), jnp.float32)

        def zrow(i, _):
            zero_v[i, 0:16] = z16
            zero_v[i, 16:32] = z16
            return 0

        lax.fori_loop(0, ROWS_PER_TILE, zrow, 0)
        pltpu.sync_copy(zero_v, acc_sh.at[pl.ds(s * ROWS_PER_TILE, ROWS_PER_TILE)])
        plsc.subcore_barrier()

        def body(j, _):
            off = base + j * CHUNK
            pltpu.sync_copy(dst_hbm.at[pl.ds(off, CHUNK)], idx_v)
            pltpu.sync_copy(msg_hbm.at[pl.ds(off, CHUNK)], msg_v)
            pltpu.sync_copy(msg_v, acc_sh.at[idx_v], add=True)
            return 0

        lax.fori_loop(0, n_chunks, body, 0)
        plsc.subcore_barrier()
        pltpu.sync_copy(
            acc_sh.at[pl.ds(s * ROWS_PER_TILE, ROWS_PER_TILE)],
            out_hbm.at[c, pl.ds(s * ROWS_PER_TILE, ROWS_PER_TILE)],
        )

    return k(msg, dst)


def _count_call(dst, ep):
    n_chunks = ep // NW // CHUNK

    @functools.partial(
        pl.kernel,
        out_type=jax.ShapeDtypeStruct((2, NPAD, 16), jnp.float32),
        mesh=_mesh(),
        compiler_params=_SC_PARAMS,
        scratch_types=[
            pltpu.VMEM((CHUNK,), jnp.int32),
            pltpu.VMEM((CHUNK, 16), jnp.float32),
            pltpu.VMEM((ROWS_PER_TILE, 16), jnp.float32),
            pltpu.VMEM_SHARED((NPAD, 16), jnp.float32),
        ],
    )
    def k(dst_hbm, out_hbm, idx_v, ones_v, zero_v, acc_sh):
        c = lax.axis_index("c")
        s = lax.axis_index("s")
        wid = s * 2 + c
        base = wid * (ep // NW)

        z16 = jnp.zeros((16,), jnp.float32)
        o16 = jnp.ones((16,), jnp.float32)

        def fill(i, _):
            zero_v[i, 0:16] = z16
            return 0

        def fill1(i, _):
            ones_v[i, 0:16] = o16
            return 0

        lax.fori_loop(0, ROWS_PER_TILE, fill, 0)
        lax.fori_loop(0, CHUNK, fill1, 0)
        pltpu.sync_copy(zero_v, acc_sh.at[pl.ds(s * ROWS_PER_TILE, ROWS_PER_TILE)])
        plsc.subcore_barrier()

        def body(j, _):
            off = base + j * CHUNK
            pltpu.sync_copy(dst_hbm.at[pl.ds(off, CHUNK)], idx_v)
            pltpu.sync_copy(ones_v, acc_sh.at[idx_v], add=True)
            return 0

        lax.fori_loop(0, n_chunks, body, 0)
        plsc.subcore_barrier()
        pltpu.sync_copy(
            acc_sh.at[pl.ds(s * ROWS_PER_TILE, ROWS_PER_TILE)],
            out_hbm.at[c, pl.ds(s * ROWS_PER_TILE, ROWS_PER_TILE)],
        )

    return k(dst)


def _edge_body(ea_ref, xs_ref, w1_ref, b1_ref, w2p_ref, bm_ref, t32_ref, s_ref,
               out_ref):
    g = jax.nn.gelu(
        jnp.dot(ea_ref[...], w1_ref[...], preferred_element_type=jnp.float32)
        + b1_ref[...]
    )
    xs = xs_ref[...]
    xt = jnp.dot(xs, t32_ref[...], preferred_element_type=jnp.float32)
    kp = jnp.dot(g, w2p_ref[...], preferred_element_type=jnp.float32)
    out_ref[...] = (
        jnp.dot(xt * kp, s_ref[...], preferred_element_type=jnp.float32)
        + jnp.dot(xs, bm_ref[...], preferred_element_type=jnp.float32)
    )


def _edge_call(ea, xs, w1p, b1, w2p, bm, t32, smat, ep):
    grid = ep // TILE_E
    return pl.pallas_call(
        _edge_body,
        grid=(grid,),
        in_specs=[
            pl.BlockSpec((TILE_E, 8), lambda i: (i, 0)),
            pl.BlockSpec((TILE_E, D), lambda i: (i, 0)),
            pl.BlockSpec((8, D), lambda i: (0, 0)),
            pl.BlockSpec((1, D), lambda i: (0, 0)),
            pl.BlockSpec((D, D * D), lambda i: (0, 0)),
            pl.BlockSpec((D, D), lambda i: (0, 0)),
            pl.BlockSpec((D, D * D), lambda i: (0, 0)),
            pl.BlockSpec((D * D, D), lambda i: (0, 0)),
        ],
        out_specs=pl.BlockSpec((TILE_E, D), lambda i: (i, 0)),
        out_shape=jax.ShapeDtypeStruct((ep, D), jnp.float32),
    )(ea, xs, w1p, b1, w2p, bm, t32, smat)


def _node_body(x_ref, parts_ref, cnt_ref, rw_ref, rb_ref, extra_ref, out_ref):
    agg = parts_ref[0] + parts_ref[1]
    cnt = cnt_ref[0, :, 0:1] + cnt_ref[1, :, 0:1]
    mean = agg / jnp.maximum(cnt, 1.0)
    h = (
        jnp.dot(x_ref[...], rw_ref[...], preferred_element_type=jnp.float32)
        + rb_ref[...]
        + mean
    )
    out_ref[...] = jax.nn.gelu(h) + extra_ref[...]


def _node_call(x, parts, cnt, rw, rb, extra):
    return pl.pallas_call(
        _node_body,
        out_shape=jax.ShapeDtypeStruct((NPAD, D), jnp.float32),
    )(x, parts, cnt, rw, rb, extra)


def _proj_body(xin_ref, w1_ref, b1_ref, w2_ref, b2_ref, out_ref):
    h = jax.nn.gelu(
        jnp.dot(xin_ref[...], w1_ref[...], preferred_element_type=jnp.float32)
        + b1_ref[...]
    )
    out_ref[...] = jax.nn.gelu(
        jnp.dot(h, w2_ref[...], preferred_element_type=jnp.float32) + b2_ref[...]
    )


def _dec_body(a_ref, b_ref, w1_ref, b1_ref, w2_ref, b2_ref, out_ref):
    h = jax.nn.gelu(
        jnp.dot(a_ref[...] + b_ref[...], w1_ref[...], preferred_element_type=jnp.float32)
        + b1_ref[...]
    )
    out_ref[...] = (
        jnp.dot(h, w2_ref[...], preferred_element_type=jnp.float32) + b2_ref[...]
    )


def _prep_edges(edge_index, edge_attr):
    e = edge_attr.shape[0]
    ep = math.ceil(e / (NW * CHUNK)) * (NW * CHUNK)
    src = jnp.concatenate(
        [edge_index[0], jnp.zeros((ep - e,), edge_index.dtype)]
    ).astype(jnp.int32)
    dst = jnp.concatenate(
        [edge_index[1], jnp.full((ep - e,), N, edge_index.dtype)]
    ).astype(jnp.int32)
    ea = jnp.pad(edge_attr, ((0, ep - e), (0, 8 - edge_attr.shape[1])))
    return src, dst, ea, ep


def _prep_gno(p):
    kp = p["kernel"]
    return dict(
        w1p=jnp.pad(kp["w1"], ((0, 8 - kp["w1"].shape[0]), (0, 0))),
        b1=kp["b1"][None, :],
        w2p=kp["w2"].reshape(D, D, D).transpose(0, 2, 1).reshape(D, D * D),
        bm=kp["b2"].reshape(D, D),
        rw=p["root_w"],
        rb=p["root_b"][None, :],
    )


def _sel_mats():
    j = lax.broadcasted_iota(jnp.int32, (D, D * D), 1)
    i = lax.broadcasted_iota(jnp.int32, (D, D * D), 0)
    t32 = (j % D == i).astype(jnp.float32)
    jj = lax.broadcasted_iota(jnp.int32, (D * D, D), 0)
    oo = lax.broadcasted_iota(jnp.int32, (D * D, D), 1)
    smat = (jj // D == oo).astype(jnp.float32)
    return t32, smat


def _gno_layer(x, src, dst, ea, ep, cnt, w, t32, smat, extra):
    xs = _gather_call(x, src, ep)
    msg = _edge_call(ea, xs, w["w1p"], w["b1"], w["w2p"], w["bm"], t32, smat, ep)
    parts = _scatter_call(msg, dst, ep)
    return _node_call(x, parts, cnt, w["rw"], w["rb"], extra)


def kernel(nodes, grid, edge_index_1, edge_index_2, edge_index_3,
           edge_attr_1, edge_attr_2, edge_attr_3, batch_size, image_size, params):
    src1, dst1, ea1, ep1 = _prep_edges(edge_index_1, edge_attr_1)
    src2, dst2, ea2, ep2 = _prep_edges(edge_index_2, edge_attr_2)
    src3, dst3, ea3, ep3 = _prep_edges(edge_index_3, edge_attr_3)

    w11 = _prep_gno(params["k11"][0])
    w12 = _prep_gno(params["k12"][0])
    w22 = _prep_gno(params["k22"][0])
    w23 = _prep_gno(params["k23"][0])
    w33 = _prep_gno(params["k33"][0])
    w32 = _prep_gno(params["k32"][0])
    w21 = _prep_gno(params["k21"][0])

    xin = jnp.pad(jnp.concatenate([nodes, grid], axis=1), ((0, NPAD - N), (0, 4)))
    pp = params["projector"]
    x0 = pl.pallas_call(
        _proj_body,
        out_shape=jax.ShapeDtypeStruct((NPAD, D), jnp.float32),
    )(
        xin,
        jnp.pad(pp["w1"], ((0, 4), (0, 0))),
        pp["b1"][None, :],
        pp["w2"],
        pp["b2"][None, :],
    )

    t32, smat = _sel_mats()
    cnt1 = _count_call(dst1, ep1)
    cnt2 = _count_call(dst2, ep2)
    cnt3 = _count_call(dst3, ep3)
    zero_extra = jnp.zeros((NPAD, D), jnp.float32)

    xs1 = _gather_call(x0, src1, ep1)
    msg11 = _edge_call(ea1, xs1, w11["w1p"], w11["b1"], w11["w2p"], w11["bm"], t32, smat, ep1)
    msg12 = _edge_call(ea1, xs1, w12["w1p"], w12["b1"], w12["w2p"], w12["bm"], t32, smat, ep1)
    n11 = _node_call(x0, _scatter_call(msg11, dst1, ep1), cnt1,
                     w11["rw"], w11["rb"], zero_extra)
    n12 = _node_call(x0, _scatter_call(msg12, dst1, ep1), cnt1,
                     w12["rw"], w12["rb"], zero_extra)

    xs2 = _gather_call(n12, src2, ep2)
    msg22 = _edge_call(ea2, xs2, w22["w1p"], w22["b1"], w22["w2p"], w22["bm"], t32, smat, ep2)
    msg23 = _edge_call(ea2, xs2, w23["w1p"], w23["b1"], w23["w2p"], w23["bm"], t32, smat, ep2)
    n22 = _node_call(n12, _scatter_call(msg22, dst2, ep2), cnt2,
                     w22["rw"], w22["rb"], zero_extra)
    n23 = _node_call(n12, _scatter_call(msg23, dst2, ep2), cnt2,
                     w23["rw"], w23["rb"], zero_extra)

    n33 = _gno_layer(n23, src3, dst3, ea3, ep3, cnt3, w33, t32, smat, zero_extra)
    s32 = _gno_layer(n33, src2, dst2, ea2, ep2, cnt2, w32, t32, smat, n22)
    n21 = _gno_layer(s32, src1, dst1, ea1, ep1, cnt1, w21, t32, smat, zero_extra)

    dp = params["decoder"]
    out = pl.pallas_call(
        _dec_body,
        out_shape=jax.ShapeDtypeStruct((NPAD, 1), jnp.float32),
    )(n21, n11, dp["w1"], dp["b1"][None, :], dp["w2"], dp["b2"][None, :])
    return out[:N]


# speedup vs baseline: 2.2225x; 1.0147x over previous
import functools
import math

import jax
import jax.numpy as jnp
from jax import lax
from jax.experimental import pallas as pl
from jax.experimental.pallas import tpu as pltpu
from jax.experimental.pallas import tpu_sc as plsc

N = 10000
NPAD = 10016
ROWS_PER_TILE = NPAD // 16
D = 32
CHUNK = 128
NW = 32
TILE_E = 512


def _mesh():
    return plsc.VectorSubcoreMesh(core_axis_name="c", subcore_axis_name="s")


_SC_PARAMS = pltpu.CompilerParams(use_tc_tiling_on_sc=False)


def _gk(rows_per_worker):
    for g in (5, 4, 2, 1):
        if rows_per_worker % g == 0:
            return g
    return 1


def _gather_call(table, idx2d, ep):
    rpw = ep // NW // CHUNK
    gk = _gk(rpw)
    n_super = rpw // gk

    @functools.partial(
        pl.kernel,
        out_type=jax.ShapeDtypeStruct((ep, D), jnp.float32),
        mesh=_mesh(),
        compiler_params=_SC_PARAMS,
        scratch_types=[
            pltpu.VMEM((gk, CHUNK), jnp.int32),
            pltpu.VMEM((gk * CHUNK, D), jnp.float32),
            pltpu.SemaphoreType.DMA,
        ],
    )
    def k(table_hbm, idx_hbm, out_hbm, idx_v, rows_v, sem):
        c = lax.axis_index("c")
        s = lax.axis_index("s")
        wid = s * 2 + c
        base = wid * rpw

        def body(j, _):
            row0 = base + j * gk
            pltpu.sync_copy(idx_hbm.at[pl.ds(row0, gk)], idx_v)
            descs = [
                pltpu.async_copy(
                    table_hbm.at[idx_v.at[b]],
                    rows_v.at[pl.ds(b * CHUNK, CHUNK)],
                    sem,
                )
                for b in range(gk)
            ]
            for d in descs:
                d.wait()
            pltpu.sync_copy(rows_v, out_hbm.at[pl.ds(row0 * CHUNK, gk * CHUNK)])
            return 0

        lax.fori_loop(0, n_super, body, 0)

    return k(table, idx2d)


def _scatter_call(msg, dst2d, ep, with_count):
    rpw = ep // NW // CHUNK
    gk = _gk(rpw)
    n_super = rpw // gk

    out_type = jax.ShapeDtypeStruct((2, NPAD, D), jnp.float32)
    scratch = [
        pltpu.VMEM((gk, CHUNK), jnp.int32),
        pltpu.VMEM((gk * CHUNK, D), jnp.float32),
        pltpu.VMEM((ROWS_PER_TILE, D), jnp.float32),
        pltpu.VMEM_SHARED((NPAD, D), jnp.float32),
        pltpu.SemaphoreType.DMA,
    ]
    if with_count:
        out_type = (out_type, jax.ShapeDtypeStruct((2, NPAD, 16), jnp.float32))
        scratch += [
            pltpu.VMEM((CHUNK, 16), jnp.float32),
            pltpu.VMEM((ROWS_PER_TILE, 16), jnp.float32),
            pltpu.VMEM_SHARED((NPAD, 16), jnp.float32),
            pltpu.SemaphoreType.DMA,
        ]

    @functools.partial(
        pl.kernel,
        out_type=out_type,
        mesh=_mesh(),
        compiler_params=_SC_PARAMS,
        scratch_types=scratch,
    )
    def k(msg_hbm, dst_hbm, *refs):
        if with_count:
            (out_hbm, cnt_hbm, idx_v, msg_v, zero_v, acc_sh, sem,
             ones_v, zcnt_v, cnt_sh, sem2) = refs
        else:
            out_hbm, idx_v, msg_v, zero_v, acc_sh, sem = refs
        c = lax.axis_index("c")
        s = lax.axis_index("s")
        wid = s * 2 + c
        base = wid * rpw

        z16 = jnp.zeros((16,), jnp.float32)

        def zrow(i, _):
            zero_v[i, 0:16] = z16
            zero_v[i, 16:32] = z16
            if with_count:
                zcnt_v[i, 0:16] = z16
            return 0

        lax.fori_loop(0, ROWS_PER_TILE, zrow, 0)
        pltpu.sync_copy(zero_v, acc_sh.at[pl.ds(s * ROWS_PER_TILE, ROWS_PER_TILE)])
        if with_count:
            o16 = jnp.ones((16,), jnp.float32)

            def orow(i, _):
                ones_v[i, 0:16] = o16
                return 0

            lax.fori_loop(0, CHUNK, orow, 0)
            pltpu.sync_copy(
                zcnt_v, cnt_sh.at[pl.ds(s * ROWS_PER_TILE, ROWS_PER_TILE)]
            )
        plsc.subcore_barrier()

        def body(j, _):
            row0 = base + j * gk
            pltpu.sync_copy(dst_hbm.at[pl.ds(row0, gk)], idx_v)
            pltpu.sync_copy(
                msg_hbm.at[pl.ds(row0 * CHUNK, gk * CHUNK)], msg_v
            )
            descs = [
                pltpu.async_copy(
                    msg_v.at[pl.ds(b * CHUNK, CHUNK)],
                    acc_sh.at[idx_v.at[b]],
                    sem,
                    add=True,
                )
                for b in range(gk)
            ]
            if with_count:
                descs += [
                    pltpu.async_copy(ones_v, cnt_sh.at[idx_v.at[b]], sem2, add=True)
                    for b in range(gk)
                ]
            for d in descs:
                d.wait()
            return 0

        lax.fori_loop(0, n_super, body, 0)
        plsc.subcore_barrier()
        pltpu.sync_copy(
            acc_sh.at[pl.ds(s * ROWS_PER_TILE, ROWS_PER_TILE)],
            out_hbm.at[c, pl.ds(s * ROWS_PER_TILE, ROWS_PER_TILE)],
        )
        if with_count:
            pltpu.sync_copy(
                cnt_sh.at[pl.ds(s * ROWS_PER_TILE, ROWS_PER_TILE)],
                cnt_hbm.at[c, pl.ds(s * ROWS_PER_TILE, ROWS_PER_TILE)],
            )

    return k(msg, dst2d)


def _edge_body(ea_ref, xs_ref, w1_ref, b1_ref, w2p_ref, bm_ref, t32_ref, s_ref,
               out_ref):
    g = jax.nn.gelu(
        jnp.dot(ea_ref[...], w1_ref[...], preferred_element_type=jnp.float32)
        + b1_ref[...]
    )
    xs = xs_ref[...]
    xt = jnp.dot(xs, t32_ref[...], preferred_element_type=jnp.float32)
    kp = jnp.dot(g, w2p_ref[...], preferred_element_type=jnp.float32)
    out_ref[...] = (
        jnp.dot(xt * kp, s_ref[...], preferred_element_type=jnp.float32)
        + jnp.dot(xs, bm_ref[...], preferred_element_type=jnp.float32)
    )


def _edge_call(ea, xs, w1p, b1, w2p, bm, t32, smat, ep):
    grid = ep // TILE_E
    return pl.pallas_call(
        _edge_body,
        grid=(grid,),
        in_specs=[
            pl.BlockSpec((TILE_E, 8), lambda i: (i, 0)),
            pl.BlockSpec((TILE_E, D), lambda i: (i, 0)),
            pl.BlockSpec((8, D), lambda i: (0, 0)),
            pl.BlockSpec((1, D), lambda i: (0, 0)),
            pl.BlockSpec((D, D * D), lambda i: (0, 0)),
            pl.BlockSpec((D, D), lambda i: (0, 0)),
            pl.BlockSpec((D, D * D), lambda i: (0, 0)),
            pl.BlockSpec((D * D, D), lambda i: (0, 0)),
        ],
        out_specs=pl.BlockSpec((TILE_E, D), lambda i: (i, 0)),
        out_shape=jax.ShapeDtypeStruct((ep, D), jnp.float32),
    )(ea, xs, w1p, b1, w2p, bm, t32, smat)


def _node_body(x_ref, parts_ref, cnt_ref, rw_ref, rb_ref, extra_ref, out_ref):
    agg = parts_ref[0] + parts_ref[1]
    cnt = cnt_ref[0, :, 0:1] + cnt_ref[1, :, 0:1]
    mean = agg / jnp.maximum(cnt, 1.0)
    h = (
        jnp.dot(x_ref[...], rw_ref[...], preferred_element_type=jnp.float32)
        + rb_ref[...]
        + mean
    )
    out_ref[...] = jax.nn.gelu(h) + extra_ref[...]


def _node_call(x, parts, cnt, rw, rb, extra):
    return pl.pallas_call(
        _node_body,
        out_shape=jax.ShapeDtypeStruct((NPAD, D), jnp.float32),
    )(x, parts, cnt, rw, rb, extra)


def _proj_body(xin_ref, w1_ref, b1_ref, w2_ref, b2_ref, out_ref):
    h = jax.nn.gelu(
        jnp.dot(xin_ref[...], w1_ref[...], preferred_element_type=jnp.float32)
        + b1_ref[...]
    )
    out_ref[...] = jax.nn.gelu(
        jnp.dot(h, w2_ref[...], preferred_element_type=jnp.float32) + b2_ref[...]
    )


def _dec_body(a_ref, b_ref, w1_ref, b1_ref, w2_ref, b2_ref, out_ref):
    h = jax.nn.gelu(
        jnp.dot(a_ref[...] + b_ref[...], w1_ref[...], preferred_element_type=jnp.float32)
        + b1_ref[...]
    )
    out_ref[...] = (
        jnp.dot(h, w2_ref[...], preferred_element_type=jnp.float32) + b2_ref[...]
    )


def _prep_edges(edge_index, edge_attr):
    e = edge_attr.shape[0]
    ep = math.ceil(e / (NW * CHUNK)) * (NW * CHUNK)
    src = jnp.concatenate(
        [edge_index[0], jnp.zeros((ep - e,), edge_index.dtype)]
    ).astype(jnp.int32).reshape(ep // CHUNK, CHUNK)
    dst = jnp.concatenate(
        [edge_index[1], jnp.full((ep - e,), N, edge_index.dtype)]
    ).astype(jnp.int32).reshape(ep // CHUNK, CHUNK)
    ea = jnp.pad(edge_attr, ((0, ep - e), (0, 8 - edge_attr.shape[1])))
    return src, dst, ea, ep


def _prep_gno(p):
    kp = p["kernel"]
    return dict(
        w1p=jnp.pad(kp["w1"], ((0, 8 - kp["w1"].shape[0]), (0, 0))),
        b1=kp["b1"][None, :],
        w2p=kp["w2"].reshape(D, D, D).transpose(0, 2, 1).reshape(D, D * D),
        bm=kp["b2"].reshape(D, D),
        rw=p["root_w"],
        rb=p["root_b"][None, :],
    )


def _sel_mats():
    j = lax.broadcasted_iota(jnp.int32, (D, D * D), 1)
    i = lax.broadcasted_iota(jnp.int32, (D, D * D), 0)
    t32 = (j % D == i).astype(jnp.float32)
    jj = lax.broadcasted_iota(jnp.int32, (D * D, D), 0)
    oo = lax.broadcasted_iota(jnp.int32, (D * D, D), 1)
    smat = (jj // D == oo).astype(jnp.float32)
    return t32, smat


def _gno_layer(x, src, dst, ea, ep, cnt, w, t32, smat, extra):
    xs = _gather_call(x, src, ep)
    msg = _edge_call(ea, xs, w["w1p"], w["b1"], w["w2p"], w["bm"], t32, smat, ep)
    parts = _scatter_call(msg, dst, ep, False)
    return _node_call(x, parts, cnt, w["rw"], w["rb"], extra)


def kernel(nodes, grid, edge_index_1, edge_index_2, edge_index_3,
           edge_attr_1, edge_attr_2, edge_attr_3, batch_size, image_size, params):
    src1, dst1, ea1, ep1 = _prep_edges(edge_index_1, edge_attr_1)
    src2, dst2, ea2, ep2 = _prep_edges(edge_index_2, edge_attr_2)
    src3, dst3, ea3, ep3 = _prep_edges(edge_index_3, edge_attr_3)

    w11 = _prep_gno(params["k11"][0])
    w12 = _prep_gno(params["k12"][0])
    w22 = _prep_gno(params["k22"][0])
    w23 = _prep_gno(params["k23"][0])
    w33 = _prep_gno(params["k33"][0])
    w32 = _prep_gno(params["k32"][0])
    w21 = _prep_gno(params["k21"][0])

    xin = jnp.pad(jnp.concatenate([nodes, grid], axis=1), ((0, NPAD - N), (0, 4)))
    pp = params["projector"]
    x0 = pl.pallas_call(
        _proj_body,
        out_shape=jax.ShapeDtypeStruct((NPAD, D), jnp.float32),
    )(
        xin,
        jnp.pad(pp["w1"], ((0, 4), (0, 0))),
        pp["b1"][None, :],
        pp["w2"],
        pp["b2"][None, :],
    )

    t32, smat = _sel_mats()
    zero_extra = jnp.zeros((NPAD, D), jnp.float32)

    xs1 = _gather_call(x0, src1, ep1)
    msg11 = _edge_call(ea1, xs1, w11["w1p"], w11["b1"], w11["w2p"], w11["bm"], t32, smat, ep1)
    msg12 = _edge_call(ea1, xs1, w12["w1p"], w12["b1"], w12["w2p"], w12["bm"], t32, smat, ep1)
    parts11, cnt1 = _scatter_call(msg11, dst1, ep1, True)
    n11 = _node_call(x0, parts11, cnt1, w11["rw"], w11["rb"], zero_extra)
    n12 = _node_call(x0, _scatter_call(msg12, dst1, ep1, False), cnt1,
                     w12["rw"], w12["rb"], zero_extra)

    xs2 = _gather_call(n12, src2, ep2)
    msg22 = _edge_call(ea2, xs2, w22["w1p"], w22["b1"], w22["w2p"], w22["bm"], t32, smat, ep2)
    msg23 = _edge_call(ea2, xs2, w23["w1p"], w23["b1"], w23["w2p"], w23["bm"], t32, smat, ep2)
    parts22, cnt2 = _scatter_call(msg22, dst2, ep2, True)
    n22 = _node_call(n12, parts22, cnt2, w22["rw"], w22["rb"], zero_extra)
    n23 = _node_call(n12, _scatter_call(msg23, dst2, ep2, False), cnt2,
                     w23["rw"], w23["rb"], zero_extra)

    xs3 = _gather_call(n23, src3, ep3)
    msg33 = _edge_call(ea3, xs3, w33["w1p"], w33["b1"], w33["w2p"], w33["bm"], t32, smat, ep3)
    parts33, cnt3 = _scatter_call(msg33, dst3, ep3, True)
    n33 = _node_call(n23, parts33, cnt3, w33["rw"], w33["rb"], zero_extra)
    s32 = _gno_layer(n33, src2, dst2, ea2, ep2, cnt2, w32, t32, smat, n22)
    n21 = _gno_layer(s32, src1, dst1, ea1, ep1, cnt1, w21, t32, smat, zero_extra)

    dp = params["decoder"]
    out = pl.pallas_call(
        _dec_body,
        out_shape=jax.ShapeDtypeStruct((NPAD, 1), jnp.float32),
    )(n21, n11, dp["w1"], dp["b1"][None, :], dp["w2"], dp["b2"][None, :])
    return out[:N]


# speedup vs baseline: 2.8764x; 1.2942x over previous
import functools
import math

import jax
import jax.numpy as jnp
from jax import lax
from jax.experimental import pallas as pl
from jax.experimental.pallas import tpu as pltpu
from jax.experimental.pallas import tpu_sc as plsc

N = 10000
NPAD = 10016
ROWS_PER_TILE = NPAD // 16
D = 32
CHUNK = 128
NW = 32
TILE_E = 1024


def _mesh():
    return plsc.VectorSubcoreMesh(core_axis_name="c", subcore_axis_name="s")


_SC_PARAMS = pltpu.CompilerParams(use_tc_tiling_on_sc=False)


def _gk(rows_per_worker):
    for g in (5, 4, 2, 1):
        if rows_per_worker % g == 0:
            return g
    return 1


def _gather_call(table, idx2d, ep):
    rpw = ep // NW // CHUNK
    gk = _gk(rpw)
    n_super = rpw // gk

    @functools.partial(
        pl.kernel,
        out_type=jax.ShapeDtypeStruct((ep, D), jnp.float32),
        mesh=_mesh(),
        compiler_params=_SC_PARAMS,
        scratch_types=[
            pltpu.VMEM((gk, CHUNK), jnp.int32),
            pltpu.VMEM((gk * CHUNK, D), jnp.float32),
            pltpu.SemaphoreType.DMA,
        ],
    )
    def k(table_hbm, idx_hbm, out_hbm, idx_v, rows_v, sem):
        c = lax.axis_index("c")
        s = lax.axis_index("s")
        wid = s * 2 + c
        base = wid * rpw

        def body(j, _):
            row0 = base + j * gk
            pltpu.sync_copy(idx_hbm.at[pl.ds(row0, gk)], idx_v)
            descs = [
                pltpu.async_copy(
                    table_hbm.at[idx_v.at[b]],
                    rows_v.at[pl.ds(b * CHUNK, CHUNK)],
                    sem,
                )
                for b in range(gk)
            ]
            for d in descs:
                d.wait()
            pltpu.sync_copy(rows_v, out_hbm.at[pl.ds(row0 * CHUNK, gk * CHUNK)])
            return 0

        lax.fori_loop(0, n_super, body, 0)

    return k(table, idx2d)


def _scatter_call(msg, dst2d, ep, with_count):
    rpw = ep // NW // CHUNK
    gk = _gk(rpw)
    n_super = rpw // gk

    out_type = jax.ShapeDtypeStruct((2, NPAD, D), jnp.float32)
    scratch = [
        pltpu.VMEM((gk, CHUNK), jnp.int32),
        pltpu.VMEM((gk * CHUNK, D), jnp.float32),
        pltpu.VMEM((ROWS_PER_TILE, D), jnp.float32),
        pltpu.VMEM_SHARED((NPAD, D), jnp.float32),
        pltpu.SemaphoreType.DMA,
    ]
    if with_count:
        out_type = (out_type, jax.ShapeDtypeStruct((2, NPAD, 16), jnp.float32))
        scratch += [
            pltpu.VMEM((CHUNK, 16), jnp.float32),
            pltpu.VMEM((ROWS_PER_TILE, 16), jnp.float32),
            pltpu.VMEM_SHARED((NPAD, 16), jnp.float32),
            pltpu.SemaphoreType.DMA,
        ]

    @functools.partial(
        pl.kernel,
        out_type=out_type,
        mesh=_mesh(),
        compiler_params=_SC_PARAMS,
        scratch_types=scratch,
    )
    def k(msg_hbm, dst_hbm, *refs):
        if with_count:
            (out_hbm, cnt_hbm, idx_v, msg_v, zero_v, acc_sh, sem,
             ones_v, zcnt_v, cnt_sh, sem2) = refs
        else:
            out_hbm, idx_v, msg_v, zero_v, acc_sh, sem = refs
        c = lax.axis_index("c")
        s = lax.axis_index("s")
        wid = s * 2 + c
        base = wid * rpw

        z16 = jnp.zeros((16,), jnp.float32)

        def zrow(i, _):
            zero_v[i, 0:16] = z16
            zero_v[i, 16:32] = z16
            if with_count:
                zcnt_v[i, 0:16] = z16
            return 0

        lax.fori_loop(0, ROWS_PER_TILE, zrow, 0)
        pltpu.sync_copy(zero_v, acc_sh.at[pl.ds(s * ROWS_PER_TILE, ROWS_PER_TILE)])
        if with_count:
            o16 = jnp.ones((16,), jnp.float32)

            def orow(i, _):
                ones_v[i, 0:16] = o16
                return 0

            lax.fori_loop(0, CHUNK, orow, 0)
            pltpu.sync_copy(
                zcnt_v, cnt_sh.at[pl.ds(s * ROWS_PER_TILE, ROWS_PER_TILE)]
            )
        plsc.subcore_barrier()

        def body(j, _):
            row0 = base + j * gk
            pltpu.sync_copy(dst_hbm.at[pl.ds(row0, gk)], idx_v)
            pltpu.sync_copy(
                msg_hbm.at[pl.ds(row0 * CHUNK, gk * CHUNK)], msg_v
            )
            descs = [
                pltpu.async_copy(
                    msg_v.at[pl.ds(b * CHUNK, CHUNK)],
                    acc_sh.at[idx_v.at[b]],
                    sem,
                    add=True,
                )
                for b in range(gk)
            ]
            if with_count:
                descs += [
                    pltpu.async_copy(ones_v, cnt_sh.at[idx_v.at[b]], sem2, add=True)
                    for b in range(gk)
                ]
            for d in descs:
                d.wait()
            return 0

        lax.fori_loop(0, n_super, body, 0)
        plsc.subcore_barrier()
        pltpu.sync_copy(
            acc_sh.at[pl.ds(s * ROWS_PER_TILE, ROWS_PER_TILE)],
            out_hbm.at[c, pl.ds(s * ROWS_PER_TILE, ROWS_PER_TILE)],
        )
        if with_count:
            pltpu.sync_copy(
                cnt_sh.at[pl.ds(s * ROWS_PER_TILE, ROWS_PER_TILE)],
                cnt_hbm.at[c, pl.ds(s * ROWS_PER_TILE, ROWS_PER_TILE)],
            )

    return k(msg, dst2d)


def _edge_body(ea_ref, xs_ref, w1_ref, b1_ref, w2p_ref, bm_ref, t32_ref, s_ref,
               out_ref):
    g = jax.nn.gelu(
        jnp.dot(ea_ref[...], w1_ref[...], preferred_element_type=jnp.float32)
        + b1_ref[...]
    )
    xs = xs_ref[...]
    xt = jnp.tile(xs, (1, D))
    kp = jnp.dot(g, w2p_ref[...], preferred_element_type=jnp.float32)
    out_ref[...] = (
        jnp.dot(xt * kp, s_ref[...], preferred_element_type=jnp.float32)
        + jnp.dot(xs, bm_ref[...], preferred_element_type=jnp.float32)
    )


def _edge_call(ea, xs, w1p, b1, w2p, bm, t32, smat, ep):
    grid = ep // TILE_E
    return pl.pallas_call(
        _edge_body,
        grid=(grid,),
        in_specs=[
            pl.BlockSpec((TILE_E, 8), lambda i: (i, 0)),
            pl.BlockSpec((TILE_E, D), lambda i: (i, 0)),
            pl.BlockSpec((8, D), lambda i: (0, 0)),
            pl.BlockSpec((1, D), lambda i: (0, 0)),
            pl.BlockSpec((D, D * D), lambda i: (0, 0)),
            pl.BlockSpec((D, D), lambda i: (0, 0)),
            pl.BlockSpec((D, D * D), lambda i: (0, 0)),
            pl.BlockSpec((D * D, D), lambda i: (0, 0)),
        ],
        out_specs=pl.BlockSpec((TILE_E, D), lambda i: (i, 0)),
        out_shape=jax.ShapeDtypeStruct((ep, D), jnp.float32),
    )(ea, xs, w1p, b1, w2p, bm, t32, smat)


def _node_body(x_ref, parts_ref, cnt_ref, rw_ref, rb_ref, extra_ref, out_ref):
    agg = parts_ref[0] + parts_ref[1]
    cnt = cnt_ref[0, :, 0:1] + cnt_ref[1, :, 0:1]
    mean = agg / jnp.maximum(cnt, 1.0)
    h = (
        jnp.dot(x_ref[...], rw_ref[...], preferred_element_type=jnp.float32)
        + rb_ref[...]
        + mean
    )
    out_ref[...] = jax.nn.gelu(h) + extra_ref[...]


def _node_call(x, parts, cnt, rw, rb, extra):
    return pl.pallas_call(
        _node_body,
        out_shape=jax.ShapeDtypeStruct((NPAD, D), jnp.float32),
    )(x, parts, cnt, rw, rb, extra)


def _proj_body(xin_ref, w1_ref, b1_ref, w2_ref, b2_ref, out_ref):
    h = jax.nn.gelu(
        jnp.dot(xin_ref[...], w1_ref[...], preferred_element_type=jnp.float32)
        + b1_ref[...]
    )
    out_ref[...] = jax.nn.gelu(
        jnp.dot(h, w2_ref[...], preferred_element_type=jnp.float32) + b2_ref[...]
    )


def _dec_body(a_ref, b_ref, w1_ref, b1_ref, w2_ref, b2_ref, out_ref):
    h = jax.nn.gelu(
        jnp.dot(a_ref[...] + b_ref[...], w1_ref[...], preferred_element_type=jnp.float32)
        + b1_ref[...]
    )
    out_ref[...] = (
        jnp.dot(h, w2_ref[...], preferred_element_type=jnp.float32) + b2_ref[...]
    )


def _prep_edges(edge_index, edge_attr):
    e = edge_attr.shape[0]
    ep = math.ceil(e / (NW * CHUNK)) * (NW * CHUNK)
    src = jnp.concatenate(
        [edge_index[0], jnp.zeros((ep - e,), edge_index.dtype)]
    ).astype(jnp.int32).reshape(ep // CHUNK, CHUNK)
    dst = jnp.concatenate(
        [edge_index[1], jnp.full((ep - e,), N, edge_index.dtype)]
    ).astype(jnp.int32).reshape(ep // CHUNK, CHUNK)
    ea = jnp.pad(edge_attr, ((0, ep - e), (0, 8 - edge_attr.shape[1])))
    return src, dst, ea, ep


def _prep_gno(p):
    kp = p["kernel"]
    return dict(
        w1p=jnp.pad(kp["w1"], ((0, 8 - kp["w1"].shape[0]), (0, 0))),
        b1=kp["b1"][None, :],
        w2p=kp["w2"].reshape(D, D, D).transpose(0, 2, 1).reshape(D, D * D),
        bm=kp["b2"].reshape(D, D),
        rw=p["root_w"],
        rb=p["root_b"][None, :],
    )


def _sel_mats():
    j = lax.broadcasted_iota(jnp.int32, (D, D * D), 1)
    i = lax.broadcasted_iota(jnp.int32, (D, D * D), 0)
    t32 = (j % D == i).astype(jnp.float32)
    jj = lax.broadcasted_iota(jnp.int32, (D * D, D), 0)
    oo = lax.broadcasted_iota(jnp.int32, (D * D, D), 1)
    smat = (jj // D == oo).astype(jnp.float32)
    return t32, smat


def _gno_layer(x, src, dst, ea, ep, cnt, w, t32, smat, extra):
    xs = _gather_call(x, src, ep)
    msg = _edge_call(ea, xs, w["w1p"], w["b1"], w["w2p"], w["bm"], t32, smat, ep)
    parts = _scatter_call(msg, dst, ep, False)
    return _node_call(x, parts, cnt, w["rw"], w["rb"], extra)


def kernel(nodes, grid, edge_index_1, edge_index_2, edge_index_3,
           edge_attr_1, edge_attr_2, edge_attr_3, batch_size, image_size, params):
    src1, dst1, ea1, ep1 = _prep_edges(edge_index_1, edge_attr_1)
    src2, dst2, ea2, ep2 = _prep_edges(edge_index_2, edge_attr_2)
    src3, dst3, ea3, ep3 = _prep_edges(edge_index_3, edge_attr_3)

    w11 = _prep_gno(params["k11"][0])
    w12 = _prep_gno(params["k12"][0])
    w22 = _prep_gno(params["k22"][0])
    w23 = _prep_gno(params["k23"][0])
    w33 = _prep_gno(params["k33"][0])
    w32 = _prep_gno(params["k32"][0])
    w21 = _prep_gno(params["k21"][0])

    xin = jnp.pad(jnp.concatenate([nodes, grid], axis=1), ((0, NPAD - N), (0, 4)))
    pp = params["projector"]
    x0 = pl.pallas_call(
        _proj_body,
        out_shape=jax.ShapeDtypeStruct((NPAD, D), jnp.float32),
    )(
        xin,
        jnp.pad(pp["w1"], ((0, 4), (0, 0))),
        pp["b1"][None, :],
        pp["w2"],
        pp["b2"][None, :],
    )

    t32, smat = _sel_mats()
    zero_extra = jnp.zeros((NPAD, D), jnp.float32)

    xs1 = _gather_call(x0, src1, ep1)
    msg11 = _edge_call(ea1, xs1, w11["w1p"], w11["b1"], w11["w2p"], w11["bm"], t32, smat, ep1)
    msg12 = _edge_call(ea1, xs1, w12["w1p"], w12["b1"], w12["w2p"], w12["bm"], t32, smat, ep1)
    parts11, cnt1 = _scatter_call(msg11, dst1, ep1, True)
    n11 = _node_call(x0, parts11, cnt1, w11["rw"], w11["rb"], zero_extra)
    n12 = _node_call(x0, _scatter_call(msg12, dst1, ep1, False), cnt1,
                     w12["rw"], w12["rb"], zero_extra)

    xs2 = _gather_call(n12, src2, ep2)
    msg22 = _edge_call(ea2, xs2, w22["w1p"], w22["b1"], w22["w2p"], w22["bm"], t32, smat, ep2)
    msg23 = _edge_call(ea2, xs2, w23["w1p"], w23["b1"], w23["w2p"], w23["bm"], t32, smat, ep2)
    parts22, cnt2 = _scatter_call(msg22, dst2, ep2, True)
    n22 = _node_call(n12, parts22, cnt2, w22["rw"], w22["rb"], zero_extra)
    n23 = _node_call(n12, _scatter_call(msg23, dst2, ep2, False), cnt2,
                     w23["rw"], w23["rb"], zero_extra)

    xs3 = _gather_call(n23, src3, ep3)
    msg33 = _edge_call(ea3, xs3, w33["w1p"], w33["b1"], w33["w2p"], w33["bm"], t32, smat, ep3)
    parts33, cnt3 = _scatter_call(msg33, dst3, ep3, True)
    n33 = _node_call(n23, parts33, cnt3, w33["rw"], w33["rb"], zero_extra)
    s32 = _gno_layer(n33, src2, dst2, ea2, ep2, cnt2, w32, t32, smat, n22)
    n21 = _gno_layer(s32, src1, dst1, ea1, ep1, cnt1, w21, t32, smat, zero_extra)

    dp = params["decoder"]
    out = pl.pallas_call(
        _dec_body,
        out_shape=jax.ShapeDtypeStruct((NPAD, 1), jnp.float32),
    )(n21, n11, dp["w1"], dp["b1"][None, :], dp["w2"], dp["b2"][None, :])
    return out[:N]
